# baseline (speedup 1.0000x reference)
import functools

import jax
import jax.numpy as jnp
from jax import lax
from jax.experimental import pallas as pl
from jax.experimental.pallas import tpu as pltpu
from jax.experimental.pallas import tpu_sc as plsc

N = 10000
E = 320000
H1 = 128
H2 = 64
D_OUT = 64

NC = 2
NS = 16
CH = 128
NCH = 80
EPT = CH * NCH
EP = EPT * NC * NS
NACC = 10240
ZROWS = 64
OCH = 80
NOCH = N // OCH

_mesh = plsc.VectorSubcoreMesh(
    core_axis_name="c", subcore_axis_name="s", num_cores=NC, num_subcores=NS)


def _make_prop(H, nbuf):
    NG = NCH // nbuf

    @functools.partial(
        pl.kernel,
        out_type=jax.ShapeDtypeStruct((NC, N, H), jnp.float32),
        mesh=_mesh,
        scratch_types=(
            [pltpu.VMEM((nbuf, 2, CH), jnp.int32),
             pltpu.VMEM((nbuf, 2, CH), jnp.int32)]
            + [pltpu.VMEM((CH, H), jnp.float32) for _ in range(nbuf)]
            + [pltpu.VMEM_SHARED((NACC, H), jnp.float32)]
            + [pltpu.SemaphoreType.DMA for _ in range(2 * nbuf + 2)]
        ),
        compiler_params=pltpu.CompilerParams(use_tc_tiling_on_sc=False),
    )
    def prop(hp_hbm, ei_hbm, zrows_hbm, out_hbm, ib0, ib1, *bufs_acc_sems):
        ibufs = [ib0, ib1]
        rows = list(bufs_acc_sems[:nbuf])
        acc = bufs_acc_sems[nbuf]
        gsem = list(bufs_acc_sems[nbuf + 1:nbuf + 1 + nbuf])
        ssem = list(bufs_acc_sems[nbuf + 1 + nbuf:nbuf + 1 + 2 * nbuf])
        isem = list(bufs_acc_sems[nbuf + 1 + 2 * nbuf:])
        c = lax.axis_index("c")
        s = lax.axis_index("s")
        wid = s * NC + c

        id0 = pltpu.async_copy(ei_hbm.at[wid].at[0], ib0, isem[0])
        id1 = pltpu.async_copy(ei_hbm.at[wid].at[1], ib1, isem[1])

        def zbody(i, carry):
            pltpu.sync_copy(zrows_hbm,
                            acc.at[pl.ds(s * (NACC // NS) + i * ZROWS, ZROWS)])
            return carry
        lax.fori_loop(0, (NACC // NS) // ZROWS, zbody, 0)
        plsc.subcore_barrier()

        def run_group(g, par, refill):
            ib = ibufs[par]
            pltpu.make_async_copy(ei_hbm.at[wid].at[0], ib, isem[par]).wait()
            gd = [pltpu.async_copy(hp_hbm.at[ib.at[j].at[0]], rows[j],
                                   gsem[j]) for j in range(nbuf)]
            sd = []
            for j in range(nbuf):
                gd[j].wait()
                sd.append(pltpu.async_copy(rows[j], acc.at[ib.at[j].at[1]],
                                           ssem[j], add=True))
            for d in sd:
                d.wait()
            if refill:
                pltpu.async_copy(ei_hbm.at[wid].at[g + 2], ib, isem[par])

        def body(q, carry):
            run_group(2 * q, 0, True)
            run_group(2 * q + 1, 1, True)
            return carry
        lax.fori_loop(0, NG // 2 - 1, body, 0)
        run_group(NG - 2, 0, False)
        run_group(NG - 1, 1, False)
        plsc.subcore_barrier()

        def obody(k, carry):
            idx = s + k * NS

            @pl.when(idx < NOCH)
            def _():
                r = idx * OCH
                pltpu.sync_copy(acc.at[pl.ds(r, OCH)],
                                rows[0].at[pl.ds(0, OCH)])
                pltpu.sync_copy(rows[0].at[pl.ds(0, OCH)],
                                out_hbm.at[c].at[pl.ds(r, OCH)])
            return carry
        lax.fori_loop(0, (NOCH + NS - 1) // NS, obody, 0)

    return prop


@functools.partial(
    pl.kernel,
    out_type=jax.ShapeDtypeStruct((NC, N, 16), jnp.float32),
    mesh=_mesh,
    scratch_types=[
        pltpu.VMEM((NCH, CH), jnp.int32),
        pltpu.VMEM((CH, 16), jnp.float32),
        pltpu.VMEM((OCH, 16), jnp.float32),
        pltpu.VMEM_SHARED((NACC, 16), jnp.float32),
        pltpu.SemaphoreType.DMA,
        pltpu.SemaphoreType.DMA,
        pltpu.SemaphoreType.DMA,
        pltpu.SemaphoreType.DMA,
    ],
    compiler_params=pltpu.CompilerParams(use_tc_tiling_on_sc=False),
)
def _deg_kernel(dst_hbm, ones_hbm, z16_hbm, out_hbm, dstall, ones, stage, acc,
                sem0, sem1, sem2, sem3):
    sems = [sem0, sem1, sem2, sem3]
    c = lax.axis_index("c")
    s = lax.axis_index("s")
    wid = s * NC + c

    pltpu.sync_copy(dst_hbm.at[wid], dstall)
    pltpu.sync_copy(ones_hbm, ones)

    def zbody(i, carry):
        pltpu.sync_copy(z16_hbm,
                        acc.at[pl.ds(s * (NACC // NS) + i * ZROWS, ZROWS)])
        return carry
    lax.fori_loop(0, (NACC // NS) // ZROWS, zbody, 0)
    plsc.subcore_barrier()

    def body(p, carry):
        sd = [pltpu.async_copy(ones, acc.at[dstall.at[p * 4 + j]], sems[j],
                               add=True) for j in range(4)]
        for d in sd:
            d.wait()
        return carry
    lax.fori_loop(0, NCH // 4, body, 0)
    plsc.subcore_barrier()

    def obody(k, carry):
        idx = s + k * NS

        @pl.when(idx < NOCH)
        def _():
            r = idx * OCH
            pltpu.sync_copy(acc.at[pl.ds(r, OCH)], stage)
            pltpu.sync_copy(stage, out_hbm.at[c].at[pl.ds(r, OCH)])
        return carry
    lax.fori_loop(0, (NOCH + NS - 1) // NS, obody, 0)


def _t1(x, W1, degp):
    def body(x_ref, w_ref, dp_ref, hp_ref, dinv_ref):
        deg = dp_ref[0][:, 0:1] + dp_ref[1][:, 0:1] + 1.0
        dinv = lax.rsqrt(deg)
        h = jnp.dot(x_ref[...], w_ref[...], preferred_element_type=jnp.float32)
        hp_ref[...] = h * dinv
        dinv_ref[...] = dinv
    return pl.pallas_call(
        body,
        out_shape=(jax.ShapeDtypeStruct((N, H1), jnp.float32),
                   jax.ShapeDtypeStruct((N, 1), jnp.float32)),
    )(x, W1, degp)


def _t_mid(p, hp, dinv, b, g, be, W, Hout):
    def body(p_ref, hp_ref, dinv_ref, b_ref, g_ref, be_ref, w_ref, out_ref):
        dinv = dinv_ref[...]
        a = dinv * (p_ref[0] + p_ref[1] + hp_ref[...]) + b_ref[...]
        m = jnp.mean(a, axis=0, keepdims=True)
        v = jnp.mean((a - m) ** 2, axis=0, keepdims=True)
        t = (a - m) * lax.rsqrt(v + 1e-5) * g_ref[...] + be_ref[...]
        t = jnp.maximum(t, 0.0)
        out_ref[...] = jnp.dot(
            t, w_ref[...], preferred_element_type=jnp.float32) * dinv
    return pl.pallas_call(
        body,
        out_shape=jax.ShapeDtypeStruct((N, Hout), jnp.float32),
    )(p, hp, dinv, b.reshape(1, -1), g.reshape(1, -1), be.reshape(1, -1), W)


def _t_final(p, hp, dinv, b):
    def body(p_ref, hp_ref, dinv_ref, b_ref, out_ref):
        out_ref[...] = (dinv_ref[...] * (p_ref[0] + p_ref[1] + hp_ref[...])
                        + b_ref[...])
    return pl.pallas_call(
        body,
        out_shape=jax.ShapeDtypeStruct((N, D_OUT), jnp.float32),
    )(p, hp, dinv, b.reshape(1, -1))


_prop128 = _make_prop(H1, 2)
_prop64 = _make_prop(H2, 4)


def kernel(x, edge_index, W1, b1, g1, be1, W2, b2, g2, be2, W3, b3):
    src = edge_index[0]
    dst = edge_index[1]
    srcp = jnp.concatenate([src, jnp.zeros((EP - E,), jnp.int32)])
    dstp = jnp.concatenate([dst, jnp.full((EP - E,), N, jnp.int32)])
    dst3 = dstp.reshape(NC * NS, NCH, CH)
    ei4 = jnp.stack([srcp.reshape(NC * NS, NCH, CH), dst3], axis=2)
    ei2 = ei4.reshape(NC * NS, NCH // 2, 2, 2, CH)
    eib4 = ei4.reshape(NC * NS, NCH // 4, 4, 2, CH)

    ones16 = jnp.ones((CH, 16), jnp.float32)
    z16 = jnp.zeros((ZROWS, 16), jnp.float32)
    z128 = jnp.zeros((ZROWS, H1), jnp.float32)
    z64 = jnp.zeros((ZROWS, H2), jnp.float32)

    degp = _deg_kernel(dst3, ones16, z16)
    hp1, dinv = _t1(x, W1, degp)
    p1 = _prop128(hp1, ei2, z128)
    hp2 = _t_mid(p1, hp1, dinv, b1, g1, be1, W2, H2)
    p2 = _prop64(hp2, eib4, z64)
    hp3 = _t_mid(p2, hp2, dinv, b2, g2, be2, W3, D_OUT)
    p3 = _prop64(hp3, eib4, z64)
    return _t_final(p3, hp3, dinv, b3)
